# Initial kernel scaffold; baseline (speedup 1.0000x reference)
import functools

import jax
import jax.numpy as jnp
from jax.experimental import pallas as pl
from jax.experimental.pallas import tpu as pltpu

N = 1024
D = 2048
E = 8
HID = 512
H2 = 2 * HID
OUT = 2048
SCALE = 12.0
EPS = 1e-5
NORM_EPS = 1e-4


def _silu(g):
    return g * jax.nn.sigmoid(g)


def _prep_body(flat_ref, lnpre_ref, rw_ref, sw12_ref, sw3_ref,
               xs_ref, gates_ref, shared_ref):
    flat = flat_ref[...]
    ms = jnp.mean(flat * flat, axis=1, keepdims=True)
    xs = flat * jax.lax.rsqrt(ms + EPS) * lnpre_ref[...]
    xs_ref[...] = xs

    xn = jnp.sqrt(jnp.sum(xs * xs, axis=1, keepdims=True))
    xq = xs / jnp.maximum(xn, NORM_EPS)
    rw = rw_ref[...]
    wn = jnp.sqrt(jnp.sum(rw * rw, axis=1, keepdims=True))
    wq = rw / jnp.maximum(wn, NORM_EPS)
    logits = jax.lax.dot_general(
        xq, wq, (((1,), (1,)), ((), ())),
        preferred_element_type=jnp.float32) * SCALE

    neg = jnp.float32(-1e30)
    m1 = jnp.full((N, 1), neg, jnp.float32)
    m2 = jnp.full((N, 1), neg, jnp.float32)
    i1 = jnp.zeros((N, 1), jnp.int32)
    i2 = jnp.zeros((N, 1), jnp.int32)
    for e in range(E):
        le = logits[:, e:e + 1]
        gt1 = le > m1
        gt2 = le > m2
        m2n = jnp.where(gt1, m1, jnp.where(gt2, le, m2))
        i2n = jnp.where(gt1, i1, jnp.where(gt2, e, i2))
        m1 = jnp.where(gt1, le, m1)
        i1 = jnp.where(gt1, e, i1)
        m2, i2 = m2n, i2n
    w1 = jax.nn.sigmoid(m1 - m2)
    w2 = 1.0 - w1
    lane = jax.lax.broadcasted_iota(jnp.int32, (N, E), 1)
    gates_ref[...] = (jnp.where(lane == i1, w1, 0.0)
                      + jnp.where(lane == i2, w2, 0.0))

    h = jax.lax.dot_general(xs, sw12_ref[...], (((1,), (1,)), ((), ())),
                            preferred_element_type=jnp.float32)
    act = _silu(h[:, :HID]) * h[:, HID:]
    shared_ref[...] = jax.lax.dot_general(
        act, sw3_ref[...], (((1,), (1,)), ((), ())),
        preferred_element_type=jnp.float32)


def _expert_body(xs_ref, gates_ref, shared_ref, ew12_ref, ew3_ref,
                 lnpost_ref, out_ref, acc_ref):
    e = pl.program_id(0)
    xs = xs_ref[...]
    h = jax.lax.dot_general(xs, ew12_ref[0], (((1,), (1,)), ((), ())),
                            preferred_element_type=jnp.float32)
    act = _silu(h[:, :HID]) * h[:, HID:]
    eout = jax.lax.dot_general(act, ew3_ref[0], (((1,), (1,)), ((), ())),
                               preferred_element_type=jnp.float32)
    lane = jax.lax.broadcasted_iota(jnp.int32, (N, E), 1)
    gate = jnp.sum(jnp.where(lane == e, gates_ref[...], 0.0),
                   axis=1, keepdims=True)
    contrib = gate * eout

    @pl.when(e == 0)
    def _():
        acc_ref[...] = shared_ref[...] + contrib

    @pl.when(jnp.logical_and(e > 0, e < E - 1))
    def _():
        acc_ref[...] += contrib

    @pl.when(e == E - 1)
    def _():
        r = acc_ref[...] + contrib
        ms = jnp.mean(r * r, axis=1, keepdims=True)
        out_ref[...] = r * jax.lax.rsqrt(ms + EPS) * lnpost_ref[...]


def _full2d(shape):
    return pl.BlockSpec(shape, lambda *_: (0,) * len(shape))


def _impl(x, ln_pre_w, ln_post_w, router_w, shared_w12, shared_w3,
          expert_w12, expert_w3, interpret):
    flat = x.reshape(N, D)
    lnpre = ln_pre_w.reshape(1, D)
    lnpost = ln_post_w.reshape(1, OUT)

    xs, gates, shared = pl.pallas_call(
        _prep_body,
        out_shape=(
            jax.ShapeDtypeStruct((N, D), jnp.float32),
            jax.ShapeDtypeStruct((N, E), jnp.float32),
            jax.ShapeDtypeStruct((N, OUT), jnp.float32),
        ),
        interpret=interpret,
    )(flat, lnpre, router_w, shared_w12, shared_w3)

    out = pl.pallas_call(
        _expert_body,
        grid=(E,),
        in_specs=[
            _full2d((N, D)),
            _full2d((N, E)),
            _full2d((N, OUT)),
            pl.BlockSpec((1, H2, D), lambda e: (e, 0, 0)),
            pl.BlockSpec((1, OUT, HID), lambda e: (e, 0, 0)),
            _full2d((1, OUT)),
        ],
        out_specs=_full2d((N, OUT)),
        out_shape=jax.ShapeDtypeStruct((N, OUT), jnp.float32),
        scratch_shapes=[pltpu.VMEM((N, OUT), jnp.float32)],
        compiler_params=pltpu.CompilerParams(
            dimension_semantics=("arbitrary",)),
        interpret=interpret,
    )(xs, gates, shared, expert_w12, expert_w3, lnpost)

    return out.reshape(1, N, OUT)


def kernel(x, ln_pre_w, ln_post_w, router_w, shared_w12, shared_w3,
           expert_w12, expert_w3):
    return _impl(x, ln_pre_w, ln_post_w, router_w, shared_w12, shared_w3,
                 expert_w12, expert_w3, False)


# speedup vs baseline: 2.0631x; 2.0631x over previous
import functools

import jax
import jax.numpy as jnp
from jax.experimental import pallas as pl
from jax.experimental.pallas import tpu as pltpu

N = 1024
D = 2048
E = 8
HID = 512
H2 = 2 * HID
OUT = 2048
SCALE = 12.0
EPS = 1e-5
NORM_EPS = 1e-4


def _silu(g):
    return g * jax.nn.sigmoid(g)


def _prep_body(flat_ref, lnpre_ref, rw_ref, sw12_ref, sw3_ref,
               xs_ref, gates_ref, shared_ref):
    flat = flat_ref[...]
    ms = jnp.mean(flat * flat, axis=1, keepdims=True)
    xs = flat * jax.lax.rsqrt(ms + EPS) * lnpre_ref[...]
    xsb = xs.astype(jnp.bfloat16)
    xs_ref[...] = xsb

    xn = jnp.sqrt(jnp.sum(xs * xs, axis=1, keepdims=True))
    xq = xs / jnp.maximum(xn, NORM_EPS)
    rw = rw_ref[...]
    wn = jnp.sqrt(jnp.sum(rw * rw, axis=1, keepdims=True))
    wq = rw / jnp.maximum(wn, NORM_EPS)
    logits = jax.lax.dot_general(
        xq, wq, (((1,), (1,)), ((), ())),
        preferred_element_type=jnp.float32) * SCALE

    neg = jnp.float32(-1e30)
    m1 = jnp.full((N, 1), neg, jnp.float32)
    m2 = jnp.full((N, 1), neg, jnp.float32)
    i1 = jnp.zeros((N, 1), jnp.int32)
    i2 = jnp.zeros((N, 1), jnp.int32)
    for e in range(E):
        le = logits[:, e:e + 1]
        gt1 = le > m1
        gt2 = le > m2
        m2n = jnp.where(gt1, m1, jnp.where(gt2, le, m2))
        i2n = jnp.where(gt1, i1, jnp.where(gt2, e, i2))
        m1 = jnp.where(gt1, le, m1)
        i1 = jnp.where(gt1, e, i1)
        m2, i2 = m2n, i2n
    w1 = jax.nn.sigmoid(m1 - m2)
    w2 = 1.0 - w1
    lane = jax.lax.broadcasted_iota(jnp.int32, (N, E), 1)
    gates_ref[...] = (jnp.where(lane == i1, w1, 0.0)
                      + jnp.where(lane == i2, w2, 0.0))

    h = jax.lax.dot_general(xsb, sw12_ref[...].astype(jnp.bfloat16),
                            (((1,), (1,)), ((), ())),
                            preferred_element_type=jnp.float32)
    act = (_silu(h[:, :HID]) * h[:, HID:]).astype(jnp.bfloat16)
    shared_ref[...] = jax.lax.dot_general(
        act, sw3_ref[...].astype(jnp.bfloat16), (((1,), (1,)), ((), ())),
        preferred_element_type=jnp.float32)


def _expert_body(xs_ref, gates_ref, shared_ref, ew12_ref, ew3_ref,
                 lnpost_ref, out_ref):
    e = pl.program_id(0)
    xs = xs_ref[...]
    h = jax.lax.dot_general(xs, ew12_ref[0].astype(jnp.bfloat16),
                            (((1,), (1,)), ((), ())),
                            preferred_element_type=jnp.float32)
    act = (_silu(h[:, :HID]) * h[:, HID:]).astype(jnp.bfloat16)
    eout = jax.lax.dot_general(act, ew3_ref[0].astype(jnp.bfloat16),
                               (((1,), (1,)), ((), ())),
                               preferred_element_type=jnp.float32)
    lane = jax.lax.broadcasted_iota(jnp.int32, (N, E), 1)
    gate = jnp.sum(jnp.where(lane == e, gates_ref[...], 0.0),
                   axis=1, keepdims=True)
    contrib = gate * eout

    @pl.when(e == 0)
    def _():
        out_ref[...] = shared_ref[...] + contrib

    @pl.when(jnp.logical_and(e > 0, e < E - 1))
    def _():
        out_ref[...] += contrib

    @pl.when(e == E - 1)
    def _():
        r = out_ref[...] + contrib
        ms = jnp.mean(r * r, axis=1, keepdims=True)
        out_ref[...] = r * jax.lax.rsqrt(ms + EPS) * lnpost_ref[...]


def _full2d(shape):
    return pl.BlockSpec(shape, lambda *_: (0,) * len(shape))


def _impl(x, ln_pre_w, ln_post_w, router_w, shared_w12, shared_w3,
          expert_w12, expert_w3, interpret):
    flat = x.reshape(N, D)
    lnpre = ln_pre_w.reshape(1, D)
    lnpost = ln_post_w.reshape(1, OUT)

    xs, gates, shared = pl.pallas_call(
        _prep_body,
        out_shape=(
            jax.ShapeDtypeStruct((N, D), jnp.bfloat16),
            jax.ShapeDtypeStruct((N, E), jnp.float32),
            jax.ShapeDtypeStruct((N, OUT), jnp.float32),
        ),
        interpret=interpret,
    )(flat, lnpre, router_w, shared_w12, shared_w3)

    out = pl.pallas_call(
        _expert_body,
        grid=(E,),
        in_specs=[
            _full2d((N, D)),
            _full2d((N, E)),
            _full2d((N, OUT)),
            pl.BlockSpec((1, H2, D), lambda e: (e, 0, 0)),
            pl.BlockSpec((1, OUT, HID), lambda e: (e, 0, 0)),
            _full2d((1, OUT)),
        ],
        out_specs=_full2d((N, OUT)),
        out_shape=jax.ShapeDtypeStruct((N, OUT), jnp.float32),
        compiler_params=pltpu.CompilerParams(
            dimension_semantics=("arbitrary",)),
        interpret=interpret,
    )(xs, gates, shared, expert_w12, expert_w3, lnpost)

    return out.reshape(1, N, OUT)


def kernel(x, ln_pre_w, ln_post_w, router_w, shared_w12, shared_w3,
           expert_w12, expert_w3):
    return _impl(x, ln_pre_w, ln_post_w, router_w, shared_w12, shared_w3,
                 expert_w12, expert_w3, False)


# speedup vs baseline: 2.1544x; 1.0443x over previous
import jax
import jax.numpy as jnp
from jax.experimental import pallas as pl
from jax.experimental.pallas import tpu as pltpu

N = 1024
D = 2048
E = 8
HID = 512
H2 = 2 * HID
OUT = 2048
SCALE = 12.0
EPS = 1e-5
NORM_EPS = 1e-4

PREP_CHUNK = 256
FIN_CHUNK = 256


def _silu(g):
    return g * jax.nn.sigmoid(g)


def _prep_body(flat_ref, lnpre_ref, rw_ref, xs_ref, gates_ref):
    rw = rw_ref[...]
    wn = jnp.sqrt(jnp.sum(rw * rw, axis=1, keepdims=True))
    wq = rw / jnp.maximum(wn, NORM_EPS)
    lnpre = lnpre_ref[...]
    for c in range(N // PREP_CHUNK):
        rows = pl.ds(c * PREP_CHUNK, PREP_CHUNK)
        flat = flat_ref[rows, :]
        ms = jnp.mean(flat * flat, axis=1, keepdims=True)
        xs = flat * jax.lax.rsqrt(ms + EPS) * lnpre
        xs_ref[rows, :] = xs.astype(jnp.bfloat16)

        xn = jnp.sqrt(jnp.sum(xs * xs, axis=1, keepdims=True))
        xq = xs / jnp.maximum(xn, NORM_EPS)
        logits = jax.lax.dot_general(xq, wq, (((1,), (1,)), ((), ())),
                                     preferred_element_type=jnp.float32) * SCALE

        neg = jnp.float32(-1e30)
        m1 = jnp.full((PREP_CHUNK, 1), neg, jnp.float32)
        m2 = jnp.full((PREP_CHUNK, 1), neg, jnp.float32)
        i1 = jnp.zeros((PREP_CHUNK, 1), jnp.int32)
        i2 = jnp.zeros((PREP_CHUNK, 1), jnp.int32)
        for j in range(E):
            le = logits[:, j:j + 1]
            gt1 = le > m1
            gt2 = le > m2
            m2n = jnp.where(gt1, m1, jnp.where(gt2, le, m2))
            i2n = jnp.where(gt1, i1, jnp.where(gt2, j, i2))
            m1 = jnp.where(gt1, le, m1)
            i1 = jnp.where(gt1, j, i1)
            m2, i2 = m2n, i2n
        w1 = jax.nn.sigmoid(m1 - m2)
        lane = jax.lax.broadcasted_iota(jnp.int32, (PREP_CHUNK, E), 1)
        gates_ref[rows, :] = (jnp.where(lane == i1, w1, 0.0)
                              + jnp.where(lane == i2, 1.0 - w1, 0.0))


def _swiglu_acc(xsb, w12_ref, w3_ref, gate, out_ref, init):
    g = jax.lax.dot_general(xsb, w12_ref[pl.ds(0, HID), :].astype(jnp.bfloat16),
                            (((1,), (1,)), ((), ())),
                            preferred_element_type=jnp.float32)
    v = jax.lax.dot_general(xsb, w12_ref[pl.ds(HID, HID), :].astype(jnp.bfloat16),
                            (((1,), (1,)), ((), ())),
                            preferred_element_type=jnp.float32)
    act = _silu(g) * v
    if gate is not None:
        act = act * gate
    actb = act.astype(jnp.bfloat16)
    for o in range(4):
        half = pl.ds(o * (OUT // 4), OUT // 4)
        w3b = w3_ref[half, :].astype(jnp.bfloat16)
        part = jax.lax.dot_general(actb, w3b, (((1,), (1,)), ((), ())),
                                   preferred_element_type=jnp.float32)
        if init:
            out_ref[:, half] = part
        else:
            out_ref[:, half] += part


def _main_body(xs_ref, gates_ref, sw12_ref, sw3_ref, ew12_ref, ew3_ref,
               lnpost_ref, out_ref):
    e = pl.program_id(0)
    xsb = xs_ref[...]

    @pl.when(e == 0)
    def _shared():
        _swiglu_acc(xsb, sw12_ref, sw3_ref, None, out_ref, init=True)

    lane = jax.lax.broadcasted_iota(jnp.int32, (N, E), 1)
    gate = jnp.sum(jnp.where(lane == e, gates_ref[...], 0.0),
                   axis=1, keepdims=True)
    _swiglu_acc(xsb, ew12_ref.at[0], ew3_ref.at[0], gate, out_ref,
                init=False)

    @pl.when(e == E - 1)
    def _fin():
        lnpost = lnpost_ref[...]
        for c in range(N // FIN_CHUNK):
            rows = pl.ds(c * FIN_CHUNK, FIN_CHUNK)
            r = out_ref[rows, :]
            ms = jnp.mean(r * r, axis=1, keepdims=True)
            out_ref[rows, :] = r * jax.lax.rsqrt(ms + EPS) * lnpost


def _full(shape):
    return pl.BlockSpec(shape, lambda *_: (0,) * len(shape))


def _impl(x, ln_pre_w, ln_post_w, router_w, shared_w12, shared_w3,
          expert_w12, expert_w3, interpret):
    flat = x.reshape(N, D)
    lnpre = ln_pre_w.reshape(1, D)
    lnpost = ln_post_w.reshape(1, OUT)

    xs, gates = pl.pallas_call(
        _prep_body,
        out_shape=(
            jax.ShapeDtypeStruct((N, D), jnp.bfloat16),
            jax.ShapeDtypeStruct((N, E), jnp.float32),
        ),
        interpret=interpret,
    )(flat, lnpre, router_w)

    out = pl.pallas_call(
        _main_body,
        grid=(E,),
        in_specs=[
            _full((N, D)),
            _full((N, E)),
            _full((H2, D)),
            _full((OUT, HID)),
            pl.BlockSpec((1, H2, D), lambda e: (e, 0, 0)),
            pl.BlockSpec((1, OUT, HID), lambda e: (e, 0, 0)),
            _full((1, OUT)),
        ],
        out_specs=_full((N, OUT)),
        out_shape=jax.ShapeDtypeStruct((N, OUT), jnp.float32),
        compiler_params=pltpu.CompilerParams(
            dimension_semantics=("arbitrary",),
            vmem_limit_bytes=66912256),
        interpret=interpret,
    )(xs, gates, shared_w12, shared_w3, expert_w12, expert_w3, lnpost)

    return out.reshape(1, N, OUT)


def kernel(x, ln_pre_w, ln_post_w, router_w, shared_w12, shared_w3,
           expert_w12, expert_w3):
    return _impl(x, ln_pre_w, ln_post_w, router_w, shared_w12, shared_w3,
                 expert_w12, expert_w3, False)


# speedup vs baseline: 2.3692x; 1.0997x over previous
import jax
import jax.numpy as jnp
from jax.experimental import pallas as pl
from jax.experimental.pallas import tpu as pltpu

N = 1024
D = 2048
E = 8
HID = 512
H2 = 2 * HID
OUT = 2048
SCALE = 12.0
EPS = 1e-5
NORM_EPS = 1e-4

PREP_CHUNK = 256
FIN_CHUNK = 256


def _silu(g):
    return g * jax.nn.sigmoid(g)


def _mono_body(flat_ref, lnpre_ref, rw_ref, sw12_ref, sw3_ref,
               ew12_ref, ew3_ref, lnpost_ref, out_ref,
               xs_ref, gates_ref, g_ref):
    e = pl.program_id(0)
    j = pl.program_id(1)

    @pl.when(jnp.logical_and(e == 0, j == 0))
    def _prep():
        rw = rw_ref[...]
        wn = jnp.sqrt(jnp.sum(rw * rw, axis=1, keepdims=True))
        wq = rw / jnp.maximum(wn, NORM_EPS)
        lnpre = lnpre_ref[...]
        for c in range(N // PREP_CHUNK):
            rows = pl.ds(c * PREP_CHUNK, PREP_CHUNK)
            flat = flat_ref[rows, :]
            ms = jnp.mean(flat * flat, axis=1, keepdims=True)
            xs = flat * jax.lax.rsqrt(ms + EPS) * lnpre
            xs_ref[rows, :] = xs.astype(jnp.bfloat16)

            xn = jnp.sqrt(jnp.sum(xs * xs, axis=1, keepdims=True))
            xq = xs / jnp.maximum(xn, NORM_EPS)
            logits = jax.lax.dot_general(
                xq, wq, (((1,), (1,)), ((), ())),
                preferred_element_type=jnp.float32) * SCALE

            neg = jnp.float32(-1e30)
            m1 = jnp.full((PREP_CHUNK, 1), neg, jnp.float32)
            m2 = jnp.full((PREP_CHUNK, 1), neg, jnp.float32)
            i1 = jnp.zeros((PREP_CHUNK, 1), jnp.int32)
            i2 = jnp.zeros((PREP_CHUNK, 1), jnp.int32)
            for t in range(E):
                le = logits[:, t:t + 1]
                gt1 = le > m1
                gt2 = le > m2
                m2n = jnp.where(gt1, m1, jnp.where(gt2, le, m2))
                i2n = jnp.where(gt1, i1, jnp.where(gt2, t, i2))
                m1 = jnp.where(gt1, le, m1)
                i1 = jnp.where(gt1, t, i1)
                m2, i2 = m2n, i2n
            w1 = jax.nn.sigmoid(m1 - m2)
            lane = jax.lax.broadcasted_iota(jnp.int32, (PREP_CHUNK, E), 1)
            gates_ref[rows, :] = (jnp.where(lane == i1, w1, 0.0)
                                  + jnp.where(lane == i2, 1.0 - w1, 0.0))

        xsb = xs_ref[...]
        gs = jax.lax.dot_general(
            xsb, sw12_ref[pl.ds(0, HID), :].astype(jnp.bfloat16),
            (((1,), (1,)), ((), ())), preferred_element_type=jnp.float32)
        vs = jax.lax.dot_general(
            xsb, sw12_ref[pl.ds(HID, HID), :].astype(jnp.bfloat16),
            (((1,), (1,)), ((), ())), preferred_element_type=jnp.float32)
        actb = (_silu(gs) * vs).astype(jnp.bfloat16)
        for o in range(4):
            cols = pl.ds(o * (OUT // 4), OUT // 4)
            w3b = sw3_ref[cols, :].astype(jnp.bfloat16)
            out_ref[:, cols] = jax.lax.dot_general(
                actb, w3b, (((1,), (1,)), ((), ())),
                preferred_element_type=jnp.float32)

    @pl.when(j == 0)
    def _gate_proj():
        g_ref[...] = jax.lax.dot_general(
            xs_ref[...], ew12_ref[0].astype(jnp.bfloat16),
            (((1,), (1,)), ((), ())), preferred_element_type=jnp.float32)

    @pl.when(j == 1)
    def _value_proj():
        v = jax.lax.dot_general(
            xs_ref[...], ew12_ref[0].astype(jnp.bfloat16),
            (((1,), (1,)), ((), ())), preferred_element_type=jnp.float32)
        lane = jax.lax.broadcasted_iota(jnp.int32, (N, E), 1)
        gate = jnp.sum(jnp.where(lane == e, gates_ref[...], 0.0),
                       axis=1, keepdims=True)
        actb = (_silu(g_ref[...]) * v * gate).astype(jnp.bfloat16)
        for o in range(4):
            cols = pl.ds(o * (OUT // 4), OUT // 4)
            w3b = ew3_ref[0, cols, :].astype(jnp.bfloat16)
            out_ref[:, cols] += jax.lax.dot_general(
                actb, w3b, (((1,), (1,)), ((), ())),
                preferred_element_type=jnp.float32)

        @pl.when(e == E - 1)
        def _fin():
            lnpost = lnpost_ref[...]
            for c in range(N // FIN_CHUNK):
                rows = pl.ds(c * FIN_CHUNK, FIN_CHUNK)
                r = out_ref[rows, :]
                ms = jnp.mean(r * r, axis=1, keepdims=True)
                out_ref[rows, :] = r * jax.lax.rsqrt(ms + EPS) * lnpost


def _full(shape):
    return pl.BlockSpec(shape, lambda *_: (0,) * len(shape))


def _impl(x, ln_pre_w, ln_post_w, router_w, shared_w12, shared_w3,
          expert_w12, expert_w3, interpret):
    flat = x.reshape(N, D)
    lnpre = ln_pre_w.reshape(1, D)
    lnpost = ln_post_w.reshape(1, OUT)

    out = pl.pallas_call(
        _mono_body,
        grid=(E, 2),
        in_specs=[
            _full((N, D)),
            _full((1, D)),
            _full((E, D)),
            _full((H2, D)),
            _full((OUT, HID)),
            pl.BlockSpec((1, HID, D), lambda e, j: (e, j, 0)),
            pl.BlockSpec((1, OUT, HID), lambda e, j: (e, 0, 0)),
            _full((1, OUT)),
        ],
        out_specs=_full((N, OUT)),
        out_shape=jax.ShapeDtypeStruct((N, OUT), jnp.float32),
        scratch_shapes=[
            pltpu.VMEM((N, D), jnp.bfloat16),
            pltpu.VMEM((N, E), jnp.float32),
            pltpu.VMEM((N, HID), jnp.float32),
        ],
        compiler_params=pltpu.CompilerParams(
            dimension_semantics=("arbitrary", "arbitrary"),
            vmem_limit_bytes=66912256),
        interpret=interpret,
    )(flat, lnpre, router_w, shared_w12, shared_w3,
      expert_w12, expert_w3, lnpost)

    return out.reshape(1, N, OUT)


def kernel(x, ln_pre_w, ln_post_w, router_w, shared_w12, shared_w3,
           expert_w12, expert_w3):
    return _impl(x, ln_pre_w, ln_post_w, router_w, shared_w12, shared_w3,
                 expert_w12, expert_w3, False)
